# baseline (speedup 1.0000x reference)
import functools

import jax
import jax.numpy as jnp
from jax import lax
from jax.experimental import pallas as pl
from jax.experimental.pallas import tpu as pltpu
from jax.experimental.pallas import tpu_sc as plsc

B = 16
C = 384
CH = C // 2
P = 256
NC, NS = 2, 16
CB = 128
CC = 8
L = 16
NG = P // L


def _rsqrt(s):
    i = plsc.bitcast(s, jnp.int32)
    i = jnp.int32(0x5F3759DF) - lax.shift_right_arithmetic(i, 1)
    y = plsc.bitcast(i, jnp.float32)
    half = s * 0.5
    for _ in range(3):
        y = y * (1.5 - half * y * y)
    return y


def _sc_body(x_hbm, pid_hbm, out_hbm, pid_v, hv, wv, buf, out_local, ssq,
             part, fac, shared_ssq):
    cid = lax.axis_index("c")
    sid = lax.axis_index("s")
    b = cid * 8 + lax.div(sid, 2)
    half = lax.rem(sid, 2)
    c0_tile = half * CB
    nchunk = (CB // CC) - half * ((CB - (CH - CB)) // CC)

    pltpu.sync_copy(pid_hbm, pid_v)

    zeros = jnp.zeros((L,), jnp.float32)
    for g in range(NG):
        ssq[pl.ds(g * L, L)] = zeros
        hw = pid_v[pl.ds(g * L, L)]
        hv[pl.ds(g * L, L)] = lax.shift_right_logical(hw, 6)
        wv[pl.ds(g * L, L)] = lax.bitwise_and(hw, 63)

    iota = lax.iota(jnp.int32, L)

    def chan_body(j, col):
        jv = jnp.full((L,), j, dtype=jnp.int32)
        colv = jnp.full((L,), col, dtype=jnp.int32)
        for g in range(NG):
            h = hv[pl.ds(g * L, L)]
            w = wv[pl.ds(g * L, L)]
            vals = plsc.load_gather(buf, [jv, h, w])
            prow = iota + (g * L)
            plsc.store_scatter(out_local, [prow, colv], vals)
            plsc.addupdate(ssq.at[pl.ds(g * L, L)], vals * vals)
        return col + 1

    def chunk_body(k, _):
        with jax.named_scope("slab_dma"):
            pltpu.sync_copy(x_hbm.at[b, pl.ds(c0_tile + k * CC, CC)], buf)
        with jax.named_scope("gather"):
            lax.fori_loop(0, CC, chan_body, k * CC)
        return 0

    lax.fori_loop(0, nchunk, chunk_body, 0)

    pltpu.sync_copy(ssq, shared_ssq.at[pl.ds(sid * P, P)])
    plsc.subcore_barrier()
    pltpu.sync_copy(shared_ssq.at[pl.ds((sid ^ 1) * P, P)], part)

    for g in range(NG):
        s_tot = ssq[pl.ds(g * L, L)] + part[pl.ds(g * L, L)]
        norm = s_tot * _rsqrt(s_tot)
        fac[pl.ds(g * L, L)] = 1.0 / (norm + 1e-7)

    ncols = CB - half * (CB - (CH - CB))

    def scale_body(col, _):
        colv = jnp.full((L,), col, dtype=jnp.int32)
        for g in range(NG):
            prow = iota + (g * L)
            f = fac[pl.ds(g * L, L)]
            vals = plsc.load_gather(out_local, [prow, colv])
            plsc.store_scatter(out_local, [prow, colv], vals * f)
        return 0

    with jax.named_scope("scale"):
        lax.fori_loop(0, ncols, scale_body, 0)

    pltpu.sync_copy(out_local,
                    out_hbm.at[pl.ds(b * P, P), pl.ds(c0_tile, CB)])


@jax.jit
def _run(x4, patch_ids):
    mesh = plsc.VectorSubcoreMesh(
        core_axis_name="c", subcore_axis_name="s",
        num_cores=NC, num_subcores=NS)
    f = pl.kernel(
        _sc_body,
        out_type=jax.ShapeDtypeStruct((B * P, 2 * CB), jnp.float32),
        mesh=mesh,
        scratch_types=[
            pltpu.VMEM((P,), jnp.int32),
            pltpu.VMEM((P,), jnp.int32),
            pltpu.VMEM((P,), jnp.int32),
            pltpu.VMEM((CC, 64, 64), jnp.float32),
            pltpu.VMEM((P, CB), jnp.float32),
            pltpu.VMEM((P,), jnp.float32),
            pltpu.VMEM((P,), jnp.float32),
            pltpu.VMEM((P,), jnp.float32),
            pltpu.VMEM_SHARED((NS * P,), jnp.float32),
        ],
        compiler_params=pltpu.CompilerParams(
            use_tc_tiling_on_sc=True, needs_layout_passes=False),
    )
    return f(x4, patch_ids)[:, :CH]


def kernel(x, num_patches, patch_ids):
    out = _run(x, patch_ids)
    return (out, patch_ids)


# speedup vs baseline: 7.4332x; 7.4332x over previous
import functools

import jax
import jax.numpy as jnp
from jax import lax
from jax.experimental import pallas as pl
from jax.experimental.pallas import tpu as pltpu
from jax.experimental.pallas import tpu_sc as plsc

B = 16
C = 384
CH = C // 2
HW = 4096
P = 256
NC, NS = 2, 16
NW = NC * NS
RPT = B * P // NW
L = 16
NV = CH // L
OPAD = 2 * 128


def _rsqrt(s):
    i = plsc.bitcast(s, jnp.int32)
    i = jnp.int32(0x5F3759DF) - lax.shift_right_arithmetic(i, 1)
    y = plsc.bitcast(i, jnp.float32)
    half = s * 0.5
    for _ in range(3):
        y = y * (1.5 - half * y * y)
    return y


def _sc_body(x_hbm, pid_hbm, out_hbm, pid_v, idx_v, rows_v, ssq_all,
             fac_all, out_local, sem):
    cid = lax.axis_index("c")
    sid = lax.axis_index("s")
    wid = cid * NS + sid
    b = lax.div(wid, 2)
    poff = lax.rem(wid, 2) * RPT

    pltpu.sync_copy(pid_hbm, pid_v)

    base = b * HW
    for k in range(RPT // L):
        pv = pid_v[pl.ds(poff + k * L, L)]
        idx_v[pl.ds(k * L, L)] = pv + base

    pltpu.async_copy(x_hbm.at[idx_v], rows_v, sem).wait()

    def row_ssq(r, _):
        v = rows_v[r, pl.ds(0, L)]
        acc = v * v
        for t in range(1, NV):
            v = rows_v[r, pl.ds(t * L, L)]
            acc = acc + v * v
        ssq_all[r] = plsc.cumsum(acc)
        return 0

    lax.fori_loop(0, RPT, row_ssq, 0)

    iota = lax.iota(jnp.int32, L)
    lane15 = jnp.full((L,), L - 1, dtype=jnp.int32)
    for g in range(RPT // L):
        sg = plsc.load_gather(ssq_all, [iota + g * L, lane15])
        norm = sg * _rsqrt(sg)
        fac_all[pl.ds(g * L, L)] = 1.0 / (norm + 1e-7)

    def grp_scale(g, _):
        fv = fac_all[pl.ds(g * L, L)]
        for l in range(L):
            f = jnp.full((L,), fv[l], dtype=jnp.float32)
            r = g * L + l
            for t in range(NV):
                out_local[r, pl.ds(t * L, L)] = \
                    rows_v[r, pl.ds(t * L, L)] * f
        return 0

    lax.fori_loop(0, RPT // L, grp_scale, 0)

    pltpu.sync_copy(out_local, out_hbm.at[pl.ds(wid * RPT, RPT)])


@jax.jit
def _run(x4, patch_ids):
    xt = jnp.transpose(x4, (0, 2, 3, 1)).reshape(B * HW, C)
    mesh = plsc.VectorSubcoreMesh(
        core_axis_name="c", subcore_axis_name="s",
        num_cores=NC, num_subcores=NS)
    f = pl.kernel(
        _sc_body,
        out_type=jax.ShapeDtypeStruct((B * P, OPAD), jnp.float32),
        mesh=mesh,
        scratch_types=[
            pltpu.VMEM((P,), jnp.int32),
            pltpu.VMEM((RPT,), jnp.int32),
            pltpu.VMEM((RPT, C), jnp.float32),
            pltpu.VMEM((RPT, L), jnp.float32),
            pltpu.VMEM((RPT,), jnp.float32),
            pltpu.VMEM((RPT, OPAD), jnp.float32),
            pltpu.SemaphoreType.DMA,
        ],
        compiler_params=pltpu.CompilerParams(
            use_tc_tiling_on_sc=True, needs_layout_passes=False),
    )
    return f(xt, patch_ids)[:, :CH]


def kernel(x, num_patches, patch_ids):
    out = _run(x, patch_ids)
    return (out, patch_ids)
